# Initial kernel scaffold; baseline (speedup 1.0000x reference)
import jax
import jax.numpy as jnp
from jax.experimental import pallas as pl


def kernel(query, reference_points, input_flatten, W_samp, b_samp, W_attn, b_attn, W_val, b_val, W_out, b_out):
    raise NotImplementedError("write your pallas kernel here")


# speedup vs baseline: 9.9885x; 9.9885x over previous
import functools

import jax
import jax.numpy as jnp
import numpy as np
from jax import lax
from jax.experimental import pallas as pl
from jax.experimental.pallas import tpu as pltpu
from jax.experimental.pallas import tpu_sc as plsc

SPATIAL = [(80, 80), (40, 40), (20, 20)]
B = 8
LQ = 300
DM = 256
NH = 8
NL = 3
NP = 4
HD = 32
NLP = NL * NP
NT = sum(h * w for h, w in SPATIAL)
ITEMS = B * NH * LQ
CH = 16
NW = 32
CHUNKS_PER_W = 38
NCHUNK = NW * CHUNKS_PER_W
ITEMS_PAD = NCHUNK * CH
ROWS_PER_CHUNK = CH * 48
TBL_ROWS = B * NT * NH


def _matmul_kern(x_ref, w_ref, b_ref, o_ref):
    o_ref[...] = (
        jnp.dot(x_ref[...], w_ref[...], preferred_element_type=jnp.float32)
        + b_ref[0]
    )


def _value_proj(x_flat, W_val, b_val):
    M = x_flat.shape[0]
    TM = 2400
    grid = (M // TM,)
    return pl.pallas_call(
        _matmul_kern,
        grid=grid,
        in_specs=[
            pl.BlockSpec((TM, DM), lambda i: (i, 0)),
            pl.BlockSpec((DM, DM), lambda i: (0, 0)),
            pl.BlockSpec((1, DM), lambda i: (0, 0)),
        ],
        out_specs=pl.BlockSpec((TM, DM), lambda i: (i, 0)),
        out_shape=jax.ShapeDtypeStruct((M, DM), jnp.float32),
    )(x_flat, W_val, b_val.reshape(1, DM))


def _out_proj(x_flat, W_out, b_out):
    M = x_flat.shape[0]
    TM = 1200
    return pl.pallas_call(
        _matmul_kern,
        grid=(M // TM,),
        in_specs=[
            pl.BlockSpec((TM, DM), lambda i: (i, 0)),
            pl.BlockSpec((DM, DM), lambda i: (0, 0)),
            pl.BlockSpec((1, DM), lambda i: (0, 0)),
        ],
        out_specs=pl.BlockSpec((TM, DM), lambda i: (i, 0)),
        out_shape=jax.ShapeDtypeStruct((M, DM), jnp.float32),
    )(x_flat, W_out, b_out.reshape(1, DM))


def _samp_kern(q_ref, rx_ref, ry_ref, wx_ref, wy_ref, wa_ref,
               bx_ref, by_ref, ba_ref, g_ref,
               cw_ref, chh_ref, cbase_ref,
               i00_ref, i01_ref, i10_ref, i11_ref,
               w00_ref, w01_ref, w10_ref, w11_ref):
    b = pl.program_id(0)
    q = q_ref[0]
    ox = jnp.dot(q, wx_ref[...], preferred_element_type=jnp.float32) + bx_ref[0]
    oy = jnp.dot(q, wy_ref[...], preferred_element_type=jnp.float32) + by_ref[0]
    al = jnp.dot(q, wa_ref[...], preferred_element_type=jnp.float32) + ba_ref[0]
    al = al - jnp.max(al, axis=-1, keepdims=True)
    e = jnp.exp(al)
    denom = jnp.dot(e, g_ref[...], preferred_element_type=jnp.float32)
    attn = e / denom

    Wl = cw_ref[0]
    Hl = chh_ref[0]
    basr = cbase_ref[0]

    ix = jnp.clip(rx_ref[0] * Wl + ox - 0.5, -1e6, 1e6)
    iy = jnp.clip(ry_ref[0] * Hl + oy - 0.5, -1e6, 1e6)
    x0 = jnp.floor(ix)
    y0 = jnp.floor(iy)
    fx = ix - x0
    fy = iy - y0
    vx0 = ((x0 >= 0.0) & (x0 < Wl)).astype(jnp.float32)
    vx1 = ((x0 + 1.0 >= 0.0) & (x0 + 1.0 < Wl)).astype(jnp.float32)
    vy0 = ((y0 >= 0.0) & (y0 < Hl)).astype(jnp.float32)
    vy1 = ((y0 + 1.0 >= 0.0) & (y0 + 1.0 < Hl)).astype(jnp.float32)
    x0c = jnp.clip(x0, 0.0, Wl - 1.0)
    x1c = jnp.clip(x0 + 1.0, 0.0, Wl - 1.0)
    y0c = jnp.clip(y0, 0.0, Hl - 1.0)
    y1c = jnp.clip(y0 + 1.0, 0.0, Hl - 1.0)
    wx0 = (1.0 - fx) * vx0
    wx1 = fx * vx1
    wy0 = (1.0 - fy) * vy0
    wy1 = fy * vy1

    browf = b.astype(jnp.float32) * float(NT * NH)
    base = browf + basr
    r00 = base + (y0c * Wl + x0c) * float(NH)
    r01 = base + (y0c * Wl + x1c) * float(NH)
    r10 = base + (y1c * Wl + x0c) * float(NH)
    r11 = base + (y1c * Wl + x1c) * float(NH)
    i00_ref[0] = r00.astype(jnp.int32)
    i01_ref[0] = r01.astype(jnp.int32)
    i10_ref[0] = r10.astype(jnp.int32)
    i11_ref[0] = r11.astype(jnp.int32)
    w00_ref[0] = attn * wy0 * wx0
    w01_ref[0] = attn * wy0 * wx1
    w10_ref[0] = attn * wy1 * wx0
    w11_ref[0] = attn * wy1 * wx1


def _samp_call(query, refx, refy, Wx, Wy, Wa, bx, by, ba, G, cw, chh, cbase):
    spec_q = pl.BlockSpec((1, LQ, DM), lambda b: (b, 0, 0))
    spec_r = pl.BlockSpec((1, LQ, 96), lambda b: (b, 0, 0))
    spec_w = pl.BlockSpec((DM, 96), lambda b: (0, 0))
    spec_v = pl.BlockSpec((1, 96), lambda b: (0, 0))
    spec_g = pl.BlockSpec((96, 96), lambda b: (0, 0))
    spec_o = pl.BlockSpec((1, LQ, 96), lambda b: (b, 0, 0))
    oshape_i = jax.ShapeDtypeStruct((B, LQ, 96), jnp.int32)
    oshape_f = jax.ShapeDtypeStruct((B, LQ, 96), jnp.float32)
    return pl.pallas_call(
        _samp_kern,
        grid=(B,),
        in_specs=[spec_q, spec_r, spec_r, spec_w, spec_w, spec_w,
                  spec_v, spec_v, spec_v, spec_g, spec_v, spec_v, spec_v],
        out_specs=[spec_o] * 4 + [spec_o] * 4,
        out_shape=[oshape_i] * 4 + [oshape_f] * 4,
    )(query, refx, refy, Wx, Wy, Wa, bx, by, ba, G, cw, chh, cbase)


_SPLAT_DNUMS = lax.GatherDimensionNumbers(
    offset_dims=(), collapsed_slice_dims=(0,), start_index_map=(0,))


def _splat(v, i):
    idx = jnp.full((16, 1), i, jnp.int32)
    return lax.gather(v, idx, _SPLAT_DNUMS, (1,),
                      mode=lax.GatherScatterMode.PROMISE_IN_BOUNDS)


def _sc_body(table_hbm, idx_hbm, wgt_hbm, out_hbm, idx_v, wgt_v, rows_v,
             out_v, sem):
    nc = 2
    wid = lax.axis_index("s") * nc + lax.axis_index("c")

    def chunk_body(k, carry):
        chunk = wid * CHUNKS_PER_W + k
        pltpu.sync_copy(idx_hbm.at[chunk], idx_v)
        pltpu.sync_copy(wgt_hbm.at[chunk], wgt_v)
        cps = []
        for j in range(6):
            cps.append(pltpu.async_copy(table_hbm.at[idx_v.at[j]],
                                        rows_v.at[pl.ds(j * 128, 128), :],
                                        sem))
        for cp in cps:
            cp.wait()
        zero = jnp.zeros((16,), jnp.float32)
        for i in range(16):
            out_v[i, pl.ds(0, 16)] = zero
            out_v[i, pl.ds(16, 16)] = zero

        def s_body(s, c2):
            w_s = wgt_v[s]
            r0 = s * 16
            for i in range(16):
                ws = _splat(w_s, i)
                v0 = rows_v[r0 + i, pl.ds(0, 16)]
                v1 = rows_v[r0 + i, pl.ds(16, 16)]
                plsc.addupdate(out_v.at[i, pl.ds(0, 16)], ws * v0)
                plsc.addupdate(out_v.at[i, pl.ds(16, 16)], ws * v1)
            return c2

        lax.fori_loop(0, 48, s_body, 0, unroll=False)
        pltpu.sync_copy(out_v, out_hbm.at[chunk])
        return carry

    lax.fori_loop(0, CHUNKS_PER_W, chunk_body, 0, unroll=False)


def _sc_gather_combine(table, idx3, wgt3):
    mesh = plsc.VectorSubcoreMesh(core_axis_name="c", subcore_axis_name="s")
    kern = functools.partial(
        pl.kernel,
        mesh=mesh,
        compiler_params=pltpu.CompilerParams(use_tc_tiling_on_sc=False),
        out_type=jax.ShapeDtypeStruct((NCHUNK, 16, 32), jnp.float32),
        scratch_types=[
            pltpu.VMEM((6, 128), jnp.int32),
            pltpu.VMEM((48, 16), jnp.float32),
            pltpu.VMEM((768, 32), jnp.float32),
            pltpu.VMEM((16, 32), jnp.float32),
            pltpu.SemaphoreType.DMA,
        ],
    )(_sc_body)
    return kern(table, idx3, wgt3)


def _lane_consts():
    wl = np.zeros(96, np.float32)
    hl = np.zeros(96, np.float32)
    basr = np.zeros(96, np.float32)
    bases = [0, 6400, 8000]
    for lane in range(96):
        h = lane // NLP
        lp = lane % NLP
        l = lp // NP
        wl[lane] = SPATIAL[l][1]
        hl[lane] = SPATIAL[l][0]
        basr[lane] = bases[l] * NH + h
    g = np.zeros((96, 96), np.float32)
    for i in range(96):
        for j in range(96):
            if i // NLP == j // NLP:
                g[i, j] = 1.0
    return wl.reshape(1, 96), hl.reshape(1, 96), basr.reshape(1, 96), g


_WL, _HL, _BASR, _G = _lane_consts()


def kernel(query, reference_points, input_flatten, W_samp, b_samp, W_attn,
           b_attn, W_val, b_val, W_out, b_out):
    val = _value_proj(input_flatten.reshape(B * NT, DM), W_val, b_val)
    table = val.reshape(TBL_ROWS, HD)

    Ws = W_samp.reshape(DM, NH, NLP, 2)
    Wx = Ws[..., 0].reshape(DM, 96)
    Wy = Ws[..., 1].reshape(DM, 96)
    bs = b_samp.reshape(NH, NLP, 2)
    bx = bs[..., 0].reshape(1, 96)
    by = bs[..., 1].reshape(1, 96)
    ba = b_attn.reshape(1, 96)

    rp = reference_points
    refx = jnp.broadcast_to(rp[:, :, None, :, None, 0],
                            (B, LQ, NH, NL, NP)).reshape(B, LQ, 96)
    refy = jnp.broadcast_to(rp[:, :, None, :, None, 1],
                            (B, LQ, NH, NL, NP)).reshape(B, LQ, 96)

    i00, i01, i10, i11, w00, w01, w10, w11 = _samp_call(
        query, refx, refy, Wx, Wy, W_attn, bx, by, ba,
        jnp.asarray(_G), jnp.asarray(_WL), jnp.asarray(_HL),
        jnp.asarray(_BASR))

    idx4 = jnp.stack([i00, i01, i10, i11], axis=-1)
    idx4 = idx4.reshape(B, LQ, NH, NLP, 4).transpose(0, 2, 1, 3, 4)
    idx48 = idx4.reshape(ITEMS, 48)
    wgt4 = jnp.stack([w00, w01, w10, w11], axis=-1)
    wgt4 = wgt4.reshape(B, LQ, NH, NLP, 4).transpose(0, 2, 1, 3, 4)
    wgt48 = wgt4.reshape(ITEMS, 48)

    pad = ITEMS_PAD - ITEMS
    idx48 = jnp.pad(idx48, ((0, pad), (0, 0)))
    wgt48 = jnp.pad(wgt48, ((0, pad), (0, 0)))
    idx3 = idx48.reshape(NCHUNK, CH, 48).transpose(0, 2, 1).reshape(
        NCHUNK, 6, 128)
    wgt3 = wgt48.reshape(NCHUNK, CH, 48).transpose(0, 2, 1)

    out_sc = _sc_gather_combine(table, idx3, wgt3)

    attn_out = out_sc.reshape(ITEMS_PAD, HD)[:ITEMS]
    attn_out = attn_out.reshape(B, NH, LQ, HD).transpose(0, 2, 1, 3)
    attn_out = attn_out.reshape(B * LQ, DM)

    out = _out_proj(attn_out, W_out, b_out)
    return out.reshape(B, LQ, DM)
